# baseline (speedup 1.0000x reference)
import functools

import jax
import jax.numpy as jnp
from jax import lax
from jax.experimental import pallas as pl
from jax.experimental.pallas import tpu as pltpu
from jax.experimental.pallas import tpu_sc as plsc

B = 16
N = 128
PE = 16
EMB = 64
ROWS = B * N * N
E = B * 512

_GPB = 2


_TKN = (((0,), (0,)), ((), ()))


def _mm_body(xT1, xT2, w1, b1, w2, b2, o1, o2):
    bc1 = b1[...].reshape(EMB, 1)
    bc2 = b2[...].reshape(EMB, 1)
    for j in range(_GPB * N):
        sl = pl.ds(j * 128, 128)
        o1[j // N, j % N] = lax.dot_general(
            w1[...], xT1[:, sl], _TKN,
            preferred_element_type=jnp.float32) + bc1
        o2[j // N, j % N] = lax.dot_general(
            w2[...], xT2[:, sl], _TKN,
            preferred_element_type=jnp.float32) + bc2


def _dense_matmuls(xT1, xT2, w1, b1, w2, b2):
    cols = _GPB * N * N
    return pl.pallas_call(
        _mm_body,
        grid=(B // _GPB,),
        in_specs=[
            pl.BlockSpec((PE, cols), lambda g: (0, g)),
            pl.BlockSpec((PE, cols), lambda g: (0, g)),
            pl.BlockSpec((PE, EMB), lambda g: (0, 0)),
            pl.BlockSpec((EMB,), lambda g: (0,)),
            pl.BlockSpec((PE, EMB), lambda g: (0, 0)),
            pl.BlockSpec((EMB,), lambda g: (0,)),
        ],
        out_specs=[
            pl.BlockSpec((_GPB, N, EMB, N), lambda g: (g, 0, 0, 0)),
            pl.BlockSpec((_GPB, N, EMB, N), lambda g: (g, 0, 0, 0)),
        ],
        out_shape=[
            jax.ShapeDtypeStruct((B, N, EMB, N), jnp.float32),
            jax.ShapeDtypeStruct((B, N, EMB, N), jnp.float32),
        ],
    )(xT1, xT2, w1, b1, w2, b2)


_NC, _NS = 2, 16
_NW = _NC * _NS
_PER_W = E // _NW
_CHUNK = 128
_NCHUNK = _PER_W // _CHUNK
_NCHUNKS_TOTAL = E // _CHUNK

_KOFF = [(k // 8) * (ROWS // 128) * 1024 + (k % 8) * 128 for k in range(PE)]


def _flat_param_view(x):
    return x.T.reshape(2, 8, ROWS // 128, 128).transpose(0, 2, 1, 3).reshape(-1)


def _index_view(ei):
    return ei.reshape(2, E // _CHUNK, _CHUNK).transpose(1, 0, 2).reshape(-1)


def _gather_rows(ei, fi, z1, z2):
    mesh = plsc.VectorSubcoreMesh(core_axis_name="c", subcore_axis_name="s")

    @functools.partial(
        pl.kernel,
        mesh=mesh,
        out_type=[
            jax.ShapeDtypeStruct((_NCHUNKS_TOTAL, PE, _CHUNK), jnp.float32),
            jax.ShapeDtypeStruct((_NCHUNKS_TOTAL, PE, _CHUNK), jnp.float32),
        ],
        scratch_types=[
            pltpu.VMEM((_CHUNK,), jnp.int32),
            pltpu.VMEM((_CHUNK,), jnp.int32),
            pltpu.VMEM((_CHUNK,), jnp.int32),
            pltpu.VMEM((_CHUNK,), jnp.int32),
            pltpu.VMEM((PE, _CHUNK), jnp.int32),
            pltpu.VMEM((PE, _CHUNK), jnp.int32),
            pltpu.VMEM((PE, _CHUNK), jnp.float32),
            pltpu.VMEM((PE, _CHUNK), jnp.float32),
            pltpu.SemaphoreType.DMA,
        ],
        compiler_params=pltpu.CompilerParams(use_tc_tiling_on_sc=False),
    )
    def k(ei_h, fi_h, z1_h, z2_h, o1_h, o2_h,
          g0_v, g1_v, fi_v, ri_v, ixf, ixr, af, ar, sem):
        wid = lax.axis_index("s") * _NC + lax.axis_index("c")
        for c in range(_NCHUNK):
            cid = wid * _NCHUNK + c
            pltpu.sync_copy(ei_h.at[pl.ds(cid * 2 * _CHUNK, _CHUNK)], g0_v)
            pltpu.sync_copy(ei_h.at[pl.ds(cid * 2 * _CHUNK + _CHUNK, _CHUNK)], g1_v)
            for t in range(_CHUNK // 16):
                s = pl.ds(t * 16, 16)
                a = g0_v[s]
                b = g1_v[s]
                bi7 = a - (a & 127)
                v = (a << 7) + b - bi7
                w = (b << 7) + a - bi7
                fi_v[s] = ((v >> 7) << 10) + (v & 127)
                ri_v[s] = ((w >> 7) << 10) + (w & 127)
            for kk in range(PE):
                for t in range(_CHUNK // 16):
                    s = pl.ds(t * 16, 16)
                    ixf[kk, s] = fi_v[s] + _KOFF[kk]
                    ixr[kk, s] = ri_v[s] + _KOFF[kk]
            cps = []
            for kk in range(PE):
                cps.append(pltpu.async_copy(z1_h.at[ixf.at[kk]], af.at[kk], sem))
                cps.append(pltpu.async_copy(z1_h.at[ixr.at[kk]], ar.at[kk], sem))
            for cp in cps:
                cp.wait()

            def row_body(r, carry):
                for t in range(_CHUNK // 16):
                    s = pl.ds(t * 16, 16)
                    af[r, s] = (af[r, s] + ar[r, s]) * 0.5
                return carry

            lax.fori_loop(0, PE, row_body, 0)
            pltpu.sync_copy(af, o1_h.at[cid])
            pltpu.sync_copy(fi_h.at[pl.ds(cid * 2 * _CHUNK, _CHUNK)], g0_v)
            pltpu.sync_copy(fi_h.at[pl.ds(cid * 2 * _CHUNK + _CHUNK, _CHUNK)], g1_v)
            for t in range(_CHUNK // 16):
                s = pl.ds(t * 16, 16)
                a = g0_v[s]
                b = g1_v[s]
                v = (a << 7) + b - (a - (a & 127))
                fi_v[s] = ((v >> 7) << 10) + (v & 127)
            for kk in range(PE):
                for t in range(_CHUNK // 16):
                    s = pl.ds(t * 16, 16)
                    ixf[kk, s] = fi_v[s] + _KOFF[kk]
            cps = []
            for kk in range(PE):
                cps.append(pltpu.async_copy(z2_h.at[ixf.at[kk]], ar.at[kk], sem))
            for cp in cps:
                cp.wait()
            pltpu.sync_copy(ar, o2_h.at[cid])

    return k(ei, fi, z1, z2)


_CPB = 16


def _edge_body(s1c, s2c, w1, b1, w2, b2, o1, o2):
    bc1 = b1[...].reshape(EMB, 1)
    bc2 = b2[...].reshape(EMB, 1)
    for j in range(_CPB):
        sl = pl.ds(j * _CHUNK, _CHUNK)
        o1[:, sl] = lax.dot_general(w1[...], s1c[j], _TKN,
                                    preferred_element_type=jnp.float32) + bc1
        o2[:, sl] = lax.dot_general(w2[...], s2c[j], _TKN,
                                    preferred_element_type=jnp.float32) + bc2


def _edge_matmuls(s1c, s2c, w1, b1, w2, b2):
    return pl.pallas_call(
        _edge_body,
        grid=(_NCHUNKS_TOTAL // _CPB,),
        in_specs=[
            pl.BlockSpec((_CPB, PE, _CHUNK), lambda c: (c, 0, 0)),
            pl.BlockSpec((_CPB, PE, _CHUNK), lambda c: (c, 0, 0)),
            pl.BlockSpec((PE, EMB), lambda c: (0, 0)),
            pl.BlockSpec((EMB,), lambda c: (0,)),
            pl.BlockSpec((PE, EMB), lambda c: (0, 0)),
            pl.BlockSpec((EMB,), lambda c: (0,)),
        ],
        out_specs=[
            pl.BlockSpec((EMB, _CPB * _CHUNK), lambda c: (0, c)),
            pl.BlockSpec((EMB, _CPB * _CHUNK), lambda c: (0, c)),
        ],
        out_shape=[
            jax.ShapeDtypeStruct((EMB, E), jnp.float32),
            jax.ShapeDtypeStruct((EMB, E), jnp.float32),
        ],
    )(s1c, s2c, w1, b1, w2, b2)


def kernel(edge_RWSE, e2e_edge_RWSE, W_enc, b_enc, W_e2e, b_e2e,
           spd_table, e2e_spd_table, batch, e_batch, edge_index,
           e2e_edge_index, spd_index, spd_lengths, e2e_spd_index,
           e2e_spd_lengths):
    o1, o2 = _dense_matmuls(edge_RWSE.T, e2e_edge_RWSE.T,
                            W_enc, b_enc, W_e2e, b_e2e)
    dense = o1.transpose(0, 1, 3, 2)
    e2e_dense = o2.transpose(0, 1, 3, 2)
    s1c, s2c = _gather_rows(
        _index_view(edge_index), _index_view(e2e_edge_index),
        _flat_param_view(edge_RWSE), _flat_param_view(e2e_edge_RWSE))
    attr1T, attr2T = _edge_matmuls(s1c, s2c, W_enc, b_enc, W_e2e, b_e2e)
    return attr1T.T, attr2T.T, dense, e2e_dense


# speedup vs baseline: 1.0040x; 1.0040x over previous
import functools

import jax
import jax.numpy as jnp
from jax import lax
from jax.experimental import pallas as pl
from jax.experimental.pallas import tpu as pltpu
from jax.experimental.pallas import tpu_sc as plsc

B = 16
N = 128
PE = 16
EMB = 64
ROWS = B * N * N
E = B * 512

_GPB = 1


_TKN = (((0,), (0,)), ((), ()))


def _mm_body(xT1, xT2, w1, b1, w2, b2, o1, o2):
    bc1 = b1[...].reshape(EMB, 1)
    bc2 = b2[...].reshape(EMB, 1)
    for j in range(_GPB * N):
        sl = pl.ds(j * 128, 128)
        o1[j // N, j % N] = lax.dot_general(
            w1[...], xT1[:, sl], _TKN,
            preferred_element_type=jnp.float32) + bc1
        o2[j // N, j % N] = lax.dot_general(
            w2[...], xT2[:, sl], _TKN,
            preferred_element_type=jnp.float32) + bc2


def _dense_matmuls(xT1, xT2, w1, b1, w2, b2):
    cols = _GPB * N * N
    return pl.pallas_call(
        _mm_body,
        grid=(B // _GPB,),
        in_specs=[
            pl.BlockSpec((PE, cols), lambda g: (0, g)),
            pl.BlockSpec((PE, cols), lambda g: (0, g)),
            pl.BlockSpec((PE, EMB), lambda g: (0, 0)),
            pl.BlockSpec((EMB,), lambda g: (0,)),
            pl.BlockSpec((PE, EMB), lambda g: (0, 0)),
            pl.BlockSpec((EMB,), lambda g: (0,)),
        ],
        out_specs=[
            pl.BlockSpec((_GPB, N, EMB, N), lambda g: (g, 0, 0, 0)),
            pl.BlockSpec((_GPB, N, EMB, N), lambda g: (g, 0, 0, 0)),
        ],
        out_shape=[
            jax.ShapeDtypeStruct((B, N, EMB, N), jnp.float32),
            jax.ShapeDtypeStruct((B, N, EMB, N), jnp.float32),
        ],
    )(xT1, xT2, w1, b1, w2, b2)


_NC, _NS = 2, 16
_NW = _NC * _NS
_PER_W = E // _NW
_CHUNK = 128
_NCHUNK = _PER_W // _CHUNK
_NCHUNKS_TOTAL = E // _CHUNK

_KOFF = [(k // 8) * (ROWS // 128) * 1024 + (k % 8) * 128 for k in range(PE)]


def _flat_param_view(x):
    return x.T.reshape(2, 8, ROWS // 128, 128).transpose(0, 2, 1, 3).reshape(-1)


def _index_view(ei):
    return ei.reshape(2, E // _CHUNK, _CHUNK).transpose(1, 0, 2).reshape(-1)


def _gather_rows(ei, fi, z1, z2):
    mesh = plsc.VectorSubcoreMesh(core_axis_name="c", subcore_axis_name="s")

    @functools.partial(
        pl.kernel,
        mesh=mesh,
        out_type=[
            jax.ShapeDtypeStruct((_NCHUNKS_TOTAL, PE, _CHUNK), jnp.float32),
            jax.ShapeDtypeStruct((_NCHUNKS_TOTAL, PE, _CHUNK), jnp.float32),
        ],
        scratch_types=[
            pltpu.VMEM((_CHUNK,), jnp.int32),
            pltpu.VMEM((_CHUNK,), jnp.int32),
            pltpu.VMEM((_CHUNK,), jnp.int32),
            pltpu.VMEM((_CHUNK,), jnp.int32),
            pltpu.VMEM((PE, _CHUNK), jnp.int32),
            pltpu.VMEM((PE, _CHUNK), jnp.int32),
            pltpu.VMEM((PE, _CHUNK), jnp.float32),
            pltpu.VMEM((PE, _CHUNK), jnp.float32),
            pltpu.SemaphoreType.DMA,
        ],
        compiler_params=pltpu.CompilerParams(use_tc_tiling_on_sc=False),
    )
    def k(ei_h, fi_h, z1_h, z2_h, o1_h, o2_h,
          g0_v, g1_v, fi_v, ri_v, ixf, ixr, af, ar, sem):
        wid = lax.axis_index("s") * _NC + lax.axis_index("c")
        for c in range(_NCHUNK):
            cid = wid * _NCHUNK + c
            pltpu.sync_copy(ei_h.at[pl.ds(cid * 2 * _CHUNK, _CHUNK)], g0_v)
            pltpu.sync_copy(ei_h.at[pl.ds(cid * 2 * _CHUNK + _CHUNK, _CHUNK)], g1_v)
            for t in range(_CHUNK // 16):
                s = pl.ds(t * 16, 16)
                a = g0_v[s]
                b = g1_v[s]
                bi7 = a - (a & 127)
                v = (a << 7) + b - bi7
                w = (b << 7) + a - bi7
                fi_v[s] = ((v >> 7) << 10) + (v & 127)
                ri_v[s] = ((w >> 7) << 10) + (w & 127)
            for kk in range(PE):
                for t in range(_CHUNK // 16):
                    s = pl.ds(t * 16, 16)
                    ixf[kk, s] = fi_v[s] + _KOFF[kk]
                    ixr[kk, s] = ri_v[s] + _KOFF[kk]
            cps = []
            for kk in range(PE):
                cps.append(pltpu.async_copy(z1_h.at[ixf.at[kk]], af.at[kk], sem))
                cps.append(pltpu.async_copy(z1_h.at[ixr.at[kk]], ar.at[kk], sem))
            for cp in cps:
                cp.wait()

            def row_body(r, carry):
                for t in range(_CHUNK // 16):
                    s = pl.ds(t * 16, 16)
                    af[r, s] = (af[r, s] + ar[r, s]) * 0.5
                return carry

            lax.fori_loop(0, PE, row_body, 0)
            pltpu.sync_copy(af, o1_h.at[cid])
            pltpu.sync_copy(fi_h.at[pl.ds(cid * 2 * _CHUNK, _CHUNK)], g0_v)
            pltpu.sync_copy(fi_h.at[pl.ds(cid * 2 * _CHUNK + _CHUNK, _CHUNK)], g1_v)
            for t in range(_CHUNK // 16):
                s = pl.ds(t * 16, 16)
                a = g0_v[s]
                b = g1_v[s]
                v = (a << 7) + b - (a - (a & 127))
                fi_v[s] = ((v >> 7) << 10) + (v & 127)
            for kk in range(PE):
                for t in range(_CHUNK // 16):
                    s = pl.ds(t * 16, 16)
                    ixf[kk, s] = fi_v[s] + _KOFF[kk]
            cps = []
            for kk in range(PE):
                cps.append(pltpu.async_copy(z2_h.at[ixf.at[kk]], ar.at[kk], sem))
            for cp in cps:
                cp.wait()
            pltpu.sync_copy(ar, o2_h.at[cid])

    return k(ei, fi, z1, z2)


_CPB = 16


def _edge_body(s1c, s2c, w1, b1, w2, b2, o1, o2):
    bc1 = b1[...].reshape(EMB, 1)
    bc2 = b2[...].reshape(EMB, 1)
    for j in range(_CPB):
        sl = pl.ds(j * _CHUNK, _CHUNK)
        o1[:, sl] = lax.dot_general(w1[...], s1c[j], _TKN,
                                    preferred_element_type=jnp.float32) + bc1
        o2[:, sl] = lax.dot_general(w2[...], s2c[j], _TKN,
                                    preferred_element_type=jnp.float32) + bc2


def _edge_matmuls(s1c, s2c, w1, b1, w2, b2):
    return pl.pallas_call(
        _edge_body,
        grid=(_NCHUNKS_TOTAL // _CPB,),
        in_specs=[
            pl.BlockSpec((_CPB, PE, _CHUNK), lambda c: (c, 0, 0)),
            pl.BlockSpec((_CPB, PE, _CHUNK), lambda c: (c, 0, 0)),
            pl.BlockSpec((PE, EMB), lambda c: (0, 0)),
            pl.BlockSpec((EMB,), lambda c: (0,)),
            pl.BlockSpec((PE, EMB), lambda c: (0, 0)),
            pl.BlockSpec((EMB,), lambda c: (0,)),
        ],
        out_specs=[
            pl.BlockSpec((EMB, _CPB * _CHUNK), lambda c: (0, c)),
            pl.BlockSpec((EMB, _CPB * _CHUNK), lambda c: (0, c)),
        ],
        out_shape=[
            jax.ShapeDtypeStruct((EMB, E), jnp.float32),
            jax.ShapeDtypeStruct((EMB, E), jnp.float32),
        ],
    )(s1c, s2c, w1, b1, w2, b2)


def kernel(edge_RWSE, e2e_edge_RWSE, W_enc, b_enc, W_e2e, b_e2e,
           spd_table, e2e_spd_table, batch, e_batch, edge_index,
           e2e_edge_index, spd_index, spd_lengths, e2e_spd_index,
           e2e_spd_lengths):
    o1, o2 = _dense_matmuls(edge_RWSE.T, e2e_edge_RWSE.T,
                            W_enc, b_enc, W_e2e, b_e2e)
    dense = o1.transpose(0, 1, 3, 2)
    e2e_dense = o2.transpose(0, 1, 3, 2)
    s1c, s2c = _gather_rows(
        _index_view(edge_index), _index_view(e2e_edge_index),
        _flat_param_view(edge_RWSE), _flat_param_view(e2e_edge_RWSE))
    attr1T, attr2T = _edge_matmuls(s1c, s2c, W_enc, b_enc, W_e2e, b_e2e)
    return attr1T.T, attr2T.T, dense, e2e_dense


# speedup vs baseline: 1.0189x; 1.0149x over previous
import functools

import jax
import jax.numpy as jnp
from jax import lax
from jax.experimental import pallas as pl
from jax.experimental.pallas import tpu as pltpu
from jax.experimental.pallas import tpu_sc as plsc

B = 16
N = 128
PE = 16
EMB = 64
ROWS = B * N * N
E = B * 512

_GPB = 1


_TKN = (((0,), (0,)), ((), ()))


def _mm_body(xT1, xT2, w1, b1, w2, b2, o1, o2):
    bc1 = b1[...].reshape(EMB, 1)
    bc2 = b2[...].reshape(EMB, 1)
    for j in range(_GPB * N):
        sl = pl.ds(j * 128, 128)
        o1[j // N, j % N] = lax.dot_general(
            w1[...], xT1[:, sl], _TKN,
            preferred_element_type=jnp.float32) + bc1
        o2[j // N, j % N] = lax.dot_general(
            w2[...], xT2[:, sl], _TKN,
            preferred_element_type=jnp.float32) + bc2


def _dense_matmuls(xT1, xT2, w1, b1, w2, b2):
    cols = _GPB * N * N
    return pl.pallas_call(
        _mm_body,
        grid=(B // _GPB,),
        in_specs=[
            pl.BlockSpec((PE, cols), lambda g: (0, g)),
            pl.BlockSpec((PE, cols), lambda g: (0, g)),
            pl.BlockSpec((PE, EMB), lambda g: (0, 0)),
            pl.BlockSpec((EMB,), lambda g: (0,)),
            pl.BlockSpec((PE, EMB), lambda g: (0, 0)),
            pl.BlockSpec((EMB,), lambda g: (0,)),
        ],
        out_specs=[
            pl.BlockSpec((_GPB, N, EMB, N), lambda g: (g, 0, 0, 0)),
            pl.BlockSpec((_GPB, N, EMB, N), lambda g: (g, 0, 0, 0)),
        ],
        out_shape=[
            jax.ShapeDtypeStruct((B, N, EMB, N), jnp.float32),
            jax.ShapeDtypeStruct((B, N, EMB, N), jnp.float32),
        ],
    )(xT1, xT2, w1, b1, w2, b2)


_NC, _NS = 2, 16
_NW = _NC * _NS
_PER_W = E // _NW
_CHUNK = 128
_NCHUNK = _PER_W // _CHUNK
_NCHUNKS_TOTAL = E // _CHUNK

_KOFF = [(k // 8) * (ROWS // 128) * 1024 + (k % 8) * 128 for k in range(PE)]


def _flat_param_view(x):
    return x.T.reshape(2, 8, ROWS // 128, 128).transpose(0, 2, 1, 3).reshape(-1)


def _index_view(ei):
    return ei.reshape(2, E // _CHUNK, _CHUNK).transpose(1, 0, 2).reshape(-1)


def _gather_rows(ei, fi, z1, z2):
    mesh = plsc.VectorSubcoreMesh(core_axis_name="c", subcore_axis_name="s")

    @functools.partial(
        pl.kernel,
        mesh=mesh,
        out_type=[
            jax.ShapeDtypeStruct((_NCHUNKS_TOTAL, PE, _CHUNK), jnp.float32),
            jax.ShapeDtypeStruct((_NCHUNKS_TOTAL, PE, _CHUNK), jnp.float32),
        ],
        scratch_types=[
            pltpu.VMEM((_CHUNK,), jnp.int32),
            pltpu.VMEM((_CHUNK,), jnp.int32),
            pltpu.VMEM((_CHUNK,), jnp.int32),
            pltpu.VMEM((_CHUNK,), jnp.int32),
            pltpu.VMEM((PE, _CHUNK), jnp.int32),
            pltpu.VMEM((PE, _CHUNK), jnp.int32),
            pltpu.VMEM((PE, _CHUNK), jnp.float32),
            pltpu.VMEM((PE, _CHUNK), jnp.float32),
            pltpu.SemaphoreType.DMA,
        ],
        compiler_params=pltpu.CompilerParams(use_tc_tiling_on_sc=False),
    )
    def k(ei_h, fi_h, z1_h, z2_h, o1_h, o2_h,
          g0_v, g1_v, fi_v, ri_v, ixf, ixr, af, ar, sem):
        wid = lax.axis_index("s") * _NC + lax.axis_index("c")
        for c in range(_NCHUNK):
            cid = wid * _NCHUNK + c
            pltpu.sync_copy(ei_h.at[pl.ds(cid * 2 * _CHUNK, _CHUNK)], g0_v)
            pltpu.sync_copy(ei_h.at[pl.ds(cid * 2 * _CHUNK + _CHUNK, _CHUNK)], g1_v)
            for t in range(_CHUNK // 16):
                s = pl.ds(t * 16, 16)
                a = g0_v[s]
                b = g1_v[s]
                bi7 = a - (a & 127)
                v = (a << 7) + b - bi7
                w = (b << 7) + a - bi7
                fi_v[s] = ((v >> 7) << 10) + (v & 127)
                ri_v[s] = ((w >> 7) << 10) + (w & 127)
            for kk in range(PE):
                for t in range(_CHUNK // 16):
                    s = pl.ds(t * 16, 16)
                    ixf[kk, s] = fi_v[s] + _KOFF[kk]
                    ixr[kk, s] = ri_v[s] + _KOFF[kk]
            cps = []
            for kk in range(PE):
                cps.append(pltpu.async_copy(z1_h.at[ixf.at[kk]], af.at[kk], sem))
                cps.append(pltpu.async_copy(z1_h.at[ixr.at[kk]], ar.at[kk], sem))
            for cp in cps:
                cp.wait()

            def row_body(r, carry):
                for t in range(_CHUNK // 16):
                    s = pl.ds(t * 16, 16)
                    af[r, s] = (af[r, s] + ar[r, s]) * 0.5
                return carry

            lax.fori_loop(0, PE, row_body, 0)
            pltpu.sync_copy(af, o1_h.at[cid])
            pltpu.sync_copy(fi_h.at[pl.ds(cid * 2 * _CHUNK, _CHUNK)], g0_v)
            pltpu.sync_copy(fi_h.at[pl.ds(cid * 2 * _CHUNK + _CHUNK, _CHUNK)], g1_v)
            for t in range(_CHUNK // 16):
                s = pl.ds(t * 16, 16)
                a = g0_v[s]
                b = g1_v[s]
                v = (a << 7) + b - (a - (a & 127))
                fi_v[s] = ((v >> 7) << 10) + (v & 127)
            for kk in range(PE):
                for t in range(_CHUNK // 16):
                    s = pl.ds(t * 16, 16)
                    ixf[kk, s] = fi_v[s] + _KOFF[kk]
            cps = []
            for kk in range(PE):
                cps.append(pltpu.async_copy(z2_h.at[ixf.at[kk]], ar.at[kk], sem))
            for cp in cps:
                cp.wait()
            pltpu.sync_copy(ar, o2_h.at[cid])

    return k(ei, fi, z1, z2)


_CPB = 32


def _edge_body(s1c, s2c, w1, b1, w2, b2, o1, o2):
    bc1 = b1[...].reshape(EMB, 1)
    bc2 = b2[...].reshape(EMB, 1)
    for j in range(_CPB):
        sl = pl.ds(j * _CHUNK, _CHUNK)
        o1[:, sl] = lax.dot_general(w1[...], s1c[j], _TKN,
                                    preferred_element_type=jnp.float32) + bc1
        o2[:, sl] = lax.dot_general(w2[...], s2c[j], _TKN,
                                    preferred_element_type=jnp.float32) + bc2


def _edge_matmuls(s1c, s2c, w1, b1, w2, b2):
    return pl.pallas_call(
        _edge_body,
        grid=(_NCHUNKS_TOTAL // _CPB,),
        in_specs=[
            pl.BlockSpec((_CPB, PE, _CHUNK), lambda c: (c, 0, 0)),
            pl.BlockSpec((_CPB, PE, _CHUNK), lambda c: (c, 0, 0)),
            pl.BlockSpec((PE, EMB), lambda c: (0, 0)),
            pl.BlockSpec((EMB,), lambda c: (0,)),
            pl.BlockSpec((PE, EMB), lambda c: (0, 0)),
            pl.BlockSpec((EMB,), lambda c: (0,)),
        ],
        out_specs=[
            pl.BlockSpec((EMB, _CPB * _CHUNK), lambda c: (0, c)),
            pl.BlockSpec((EMB, _CPB * _CHUNK), lambda c: (0, c)),
        ],
        out_shape=[
            jax.ShapeDtypeStruct((EMB, E), jnp.float32),
            jax.ShapeDtypeStruct((EMB, E), jnp.float32),
        ],
    )(s1c, s2c, w1, b1, w2, b2)


def kernel(edge_RWSE, e2e_edge_RWSE, W_enc, b_enc, W_e2e, b_e2e,
           spd_table, e2e_spd_table, batch, e_batch, edge_index,
           e2e_edge_index, spd_index, spd_lengths, e2e_spd_index,
           e2e_spd_lengths):
    o1, o2 = _dense_matmuls(edge_RWSE.T, e2e_edge_RWSE.T,
                            W_enc, b_enc, W_e2e, b_e2e)
    dense = o1.transpose(0, 1, 3, 2)
    e2e_dense = o2.transpose(0, 1, 3, 2)
    s1c, s2c = _gather_rows(
        _index_view(edge_index), _index_view(e2e_edge_index),
        _flat_param_view(edge_RWSE), _flat_param_view(e2e_edge_RWSE))
    attr1T, attr2T = _edge_matmuls(s1c, s2c, W_enc, b_enc, W_e2e, b_e2e)
    return attr1T.T, attr2T.T, dense, e2e_dense
